# Initial kernel scaffold; baseline (speedup 1.0000x reference)
import jax
import jax.numpy as jnp
from jax.experimental import pallas as pl


def kernel(variables, lits, W_neg, b_neg, W1v, b1v, W2v, b2v, W1c, b1c, W2c, b2c):
    raise NotImplementedError("write your pallas kernel here")


# speedup vs baseline: 3.6699x; 3.6699x over previous
import functools

import jax
import jax.numpy as jnp
from jax import lax
from jax.experimental import pallas as pl
from jax.experimental.pallas import tpu as pltpu
from jax.experimental.pallas import tpu_sc as plsc

N = 10000
C = 8
D = 128
B = N * C * 2
ROWS_PER_CHUNK = 128
NCHUNK = B // ROWS_PER_CHUNK
NW = 32
CPW = 40

_PREP_TN = 2000


def _prep_body(vars_ref, wneg_ref, bneg_ref, lits_ref, neg_ref, idx_ref):
    v = vars_ref[...]
    neg_ref[...] = (
        jnp.dot(v, wneg_ref[...], preferred_element_type=jnp.float32)
        + bneg_ref[...]
    )
    l = lits_ref[...]
    idx_ref[...] = jnp.where(l < 0, N - 1 - l, l - 1)


def _prep(variables, W_neg, b_neg, lits12):
    nb = N // _PREP_TN
    return pl.pallas_call(
        _prep_body,
        grid=(nb,),
        in_specs=[
            pl.BlockSpec((_PREP_TN, D), lambda i: (i, 0)),
            pl.BlockSpec((D, D), lambda i: (0, 0)),
            pl.BlockSpec((1, D), lambda i: (0, 0)),
            pl.BlockSpec((_PREP_TN, 2 * C), lambda i: (i, 0)),
        ],
        out_specs=[
            pl.BlockSpec((_PREP_TN, D), lambda i: (i, 0)),
            pl.BlockSpec((_PREP_TN, 2 * C), lambda i: (i, 0)),
        ],
        out_shape=[
            jax.ShapeDtypeStruct((N, D), jnp.float32),
            jax.ShapeDtypeStruct((N, 2 * C), jnp.int32),
        ],
    )(variables, W_neg, b_neg, lits12)


def _make_gather():
    mesh = plsc.VectorSubcoreMesh(core_axis_name="c", subcore_axis_name="s")

    @functools.partial(
        pl.kernel,
        mesh=mesh,
        out_type=jax.ShapeDtypeStruct((B, D), jnp.float32),
        scratch_types=[
            pltpu.VMEM((CPW, ROWS_PER_CHUNK), jnp.int32),
            pltpu.VMEM((ROWS_PER_CHUNK, D), jnp.float32),
            pltpu.SemaphoreType.DMA,
        ],
    )
    def gather_k(table_hbm, idx_hbm, out_hbm, idx_v, rows_v, sem):
        wid = lax.axis_index("s") * 2 + lax.axis_index("c")
        base = wid * CPW
        pltpu.sync_copy(idx_hbm.at[pl.ds(base, CPW)], idx_v)

        def body(k, carry):
            c = base + k

            @pl.when(c < NCHUNK)
            def _do():
                pltpu.async_copy(table_hbm.at[idx_v.at[k]], rows_v, sem).wait()
                pltpu.sync_copy(
                    rows_v, out_hbm.at[pl.ds(c * ROWS_PER_CHUNK, ROWS_PER_CHUNK)]
                )

            return carry

        lax.fori_loop(0, CPW, body, 0)

    return gather_k


_COMB_TN = 400


def _comb_body(vars_ref, g_ref, w1v0, w1v12, w2v0, w2v12, b1v, b2v,
               w1c, w2c, b1c, b2c, out_ref):
    v = vars_ref[...]
    p10 = jnp.dot(v, w1v0[...], preferred_element_type=jnp.float32) + b1v[...]
    p20 = jnp.dot(v, w2v0[...], preferred_element_type=jnp.float32) + b2v[...]
    g = g_ref[...]
    p112 = jnp.dot(g, w1v12[...], preferred_element_type=jnp.float32)
    p212 = jnp.dot(g, w2v12[...], preferred_element_type=jnp.float32)
    pre1 = p112.reshape(_COMB_TN, C, D) + p10[:, None, :]
    pre2 = p212.reshape(_COMB_TN, C, D) + p20[:, None, :]
    cl = jax.nn.sigmoid(pre1) + pre2
    norm = jnp.sqrt(jnp.sum(cl * cl, axis=-1, keepdims=True))
    cl = cl / jnp.maximum(norm, 1e-12)
    acc1 = jnp.broadcast_to(b1c[...], (_COMB_TN, D))
    acc2 = jnp.broadcast_to(b2c[...], (_COMB_TN, D))
    for c in range(C):
        cc = cl[:, c, :]
        acc1 = acc1 + jnp.dot(cc, w1c[c], preferred_element_type=jnp.float32)
        acc2 = acc2 + jnp.dot(cc, w2c[c], preferred_element_type=jnp.float32)
    o = jax.nn.sigmoid(acc1) + acc2
    norm2 = jnp.sqrt(jnp.sum(o * o, axis=-1, keepdims=True))
    out_ref[...] = o / jnp.maximum(norm2, 1e-12)


def _combine(variables, Gv, W1v0, W1v12, W2v0, W2v12, b1v, b2v,
             W1c_r, W2c_r, b1c, b2c):
    nb = N // _COMB_TN
    return pl.pallas_call(
        _comb_body,
        grid=(nb,),
        in_specs=[
            pl.BlockSpec((_COMB_TN, D), lambda i: (i, 0)),
            pl.BlockSpec((_COMB_TN * C, 2 * D), lambda i: (i, 0)),
            pl.BlockSpec((D, D), lambda i: (0, 0)),
            pl.BlockSpec((2 * D, D), lambda i: (0, 0)),
            pl.BlockSpec((D, D), lambda i: (0, 0)),
            pl.BlockSpec((2 * D, D), lambda i: (0, 0)),
            pl.BlockSpec((1, D), lambda i: (0, 0)),
            pl.BlockSpec((1, D), lambda i: (0, 0)),
            pl.BlockSpec((C, D, D), lambda i: (0, 0, 0)),
            pl.BlockSpec((C, D, D), lambda i: (0, 0, 0)),
            pl.BlockSpec((1, D), lambda i: (0, 0)),
            pl.BlockSpec((1, D), lambda i: (0, 0)),
        ],
        out_specs=pl.BlockSpec((_COMB_TN, D), lambda i: (i, 0)),
        out_shape=jax.ShapeDtypeStruct((N, D), jnp.float32),
    )(variables, Gv, W1v0, W1v12, W2v0, W2v12, b1v, b2v,
      W1c_r, W2c_r, b1c, b2c)


def kernel(variables, lits, W_neg, b_neg, W1v, b1v, W2v, b2v,
           W1c, b1c, W2c, b2c):
    lits12 = lits[:, :, 1:].astype(jnp.int32).reshape(N, 2 * C)
    negtable, idx = _prep(variables, W_neg, b_neg.reshape(1, D), lits12)
    table = jnp.concatenate([variables, negtable], axis=0)
    idx_pad = jnp.pad(idx.reshape(-1), (0, NW * CPW * ROWS_PER_CHUNK - B))
    G = _make_gather()(table, idx_pad.reshape(NW * CPW, ROWS_PER_CHUNK))
    Gv = G.reshape(N * C, 2 * D)
    return _combine(
        variables, Gv,
        W1v[:D], W1v[D:], W2v[:D], W2v[D:],
        b1v.reshape(1, D), b2v.reshape(1, D),
        W1c.reshape(C, D, D), W2c.reshape(C, D, D),
        b1c.reshape(1, D), b2c.reshape(1, D),
    )
